# baseline (speedup 1.0000x reference)
import functools
import numpy as np
import jax
import jax.numpy as jnp
from jax import lax
from jax.experimental import pallas as pl
from jax.experimental.pallas import tpu as pltpu
from jax.experimental.pallas import tpu_sc as plsc

B = 2
D = 256
M = 8
L = 4
P = 4
DH = 32
SHAPES = ((128, 128), (64, 64), (32, 32), (16, 16))
LEN = sum(h * w for h, w in SHAPES)
ROWS = B * LEN
T = 256
NT = ROWS // T
TPB = LEN // T
NWORK = 32
RPW = ROWS // NWORK

_lanes = np.arange(M * L * P)
_m_ln = _lanes // (L * P)
_l_ln = (_lanes % (L * P)) // P
_W_np = np.array([w for h, w in SHAPES], np.float32)
_H_np = np.array([h for h, w in SHAPES], np.float32)
_start_np = np.concatenate([[0], np.cumsum([h * w for h, w in SHAPES[:-1]])]).astype(np.int64)

_WL_I = _W_np[_l_ln].astype(np.int32).reshape(1, 128)
_BASEP = (_m_ln.astype(np.int64) * LEN + _start_np[_l_ln]).astype(np.int32).reshape(1, 128)
_WM2 = (_W_np[_l_ln] - 2.0).reshape(1, 128).astype(np.float32)
_HM1 = (_H_np[_l_ln] - 1.0).reshape(1, 128).astype(np.float32)

_SX = np.zeros((8, 128), np.float32)
_SY = np.zeros((8, 128), np.float32)
for _ln in range(128):
    _SX[2 * _l_ln[_ln] + 0, _ln] = _W_np[_l_ln[_ln]]
    _SY[2 * _l_ln[_ln] + 1, _ln] = _H_np[_l_ln[_ln]]

_BLK = (_lanes[:, None] // 16 == _lanes[None, :] // 16).astype(np.float32)

_t = np.arange(32)
_ilv = np.where(_t % 2 == 0, _t // 2, 16 + _t // 2)
_c = np.arange(D)
_PERM_VAL = (_c // 32) * 32 + _ilv[_c % 32]


def _prep_body(q_ref, x_ref, rp_ref, wval_ref, bval_ref, woffx_ref, woffy_ref,
               cbx_ref, cby_ref, wattn_ref, battn_ref, sx_ref, sy_ref, blk_ref,
               wl_ref, basep_ref, wm2_ref, hm1_ref,
               val_out, idx_out, w_out):
    q = q_ref[...]
    v = jnp.dot(x_ref[...], wval_ref[...], preferred_element_type=jnp.float32) + bval_ref[...]
    val_out[...] = v.astype(jnp.bfloat16)

    hi = jax.lax.Precision.HIGHEST
    gx = (jnp.dot(q, woffx_ref[...], preferred_element_type=jnp.float32, precision=hi)
          + jnp.dot(rp_ref[...], sx_ref[...], preferred_element_type=jnp.float32, precision=hi)
          + cbx_ref[...])
    gy = (jnp.dot(q, woffy_ref[...], preferred_element_type=jnp.float32, precision=hi)
          + jnp.dot(rp_ref[...], sy_ref[...], preferred_element_type=jnp.float32, precision=hi)
          + cby_ref[...])

    a = jnp.dot(q, wattn_ref[...], preferred_element_type=jnp.float32, precision=hi) + battn_ref[...]
    e = jnp.exp(a - jnp.max(a, axis=1, keepdims=True))
    aw = e / jnp.dot(e, blk_ref[...], preferred_element_type=jnp.float32)

    x0 = jnp.floor(gx)
    y0 = jnp.floor(gy)
    y1 = y0 + 1.0
    fx = gx - x0
    fy = gy - y0

    wm2 = wm2_ref[...]
    hm1 = hm1_ref[...]
    x0c = jnp.clip(x0, 0.0, wm2)
    y0c = jnp.clip(y0, 0.0, hm1)
    y1c = jnp.clip(y1, 0.0, hm1)

    wxs0 = jnp.where(x0c == x0, 1.0 - fx, jnp.where(x0c == x0 + 1.0, fx, 0.0))
    wxs1 = jnp.where(x0c == x0, fx, jnp.where(x0c == x0 - 1.0, 1.0 - fx, 0.0))
    wy0 = jnp.where((y0 >= 0.0) & (y0 <= hm1), (1.0 - fy), 0.0) * aw
    wy1 = jnp.where((y1 >= 0.0) & (y1 <= hm1), fy, 0.0) * aw

    b = pl.program_id(0) // TPB
    boff = (b * (M * LEN)).astype(jnp.int32)
    kbase = basep_ref[...] + boff + x0c.astype(jnp.int32)
    wl = wl_ref[...]
    k0 = kbase + y0c.astype(jnp.int32) * wl
    k1 = kbase + y1c.astype(jnp.int32) * wl

    idx_out[...] = jnp.concatenate([k0, k1], axis=1)
    w_out[...] = jnp.concatenate([wy0 * wxs0, wy1 * wxs0, wy0 * wxs1, wy1 * wxs1], axis=1)


def _prep(q2, x2, rp2, W_val, bval2, W_offx, W_offy, cbx, cby, W_attn, battn2,
          sx, sy, blk, wl, basep, wm2, hm1):
    row_spec = lambda cols: pl.BlockSpec((T, cols), lambda i: (i, 0))
    full_spec = lambda r, c: pl.BlockSpec((r, c), lambda i: (0, 0))
    return pl.pallas_call(
        _prep_body,
        grid=(NT,),
        in_specs=[
            row_spec(D), row_spec(D), row_spec(8),
            full_spec(D, D), full_spec(1, D),
            full_spec(D, 128), full_spec(D, 128),
            full_spec(1, 128), full_spec(1, 128),
            full_spec(D, 128), full_spec(1, 128),
            full_spec(8, 128), full_spec(8, 128), full_spec(128, 128),
            full_spec(1, 128), full_spec(1, 128), full_spec(1, 128), full_spec(1, 128),
        ],
        out_specs=[row_spec(D), row_spec(256), row_spec(512)],
        out_shape=[
            jax.ShapeDtypeStruct((ROWS, D), jnp.bfloat16),
            jax.ShapeDtypeStruct((ROWS, 256), jnp.int32),
            jax.ShapeDtypeStruct((ROWS, 512), jnp.float32),
        ],
    )(q2, x2, rp2, W_val, bval2, W_offx, W_offy, cbx, cby, W_attn, battn2,
      sx, sy, blk, wl, basep, wm2, hm1)


_ESEL = np.zeros((M, D, DH), np.float32)
for _m in range(M):
    _ESEL[_m, _m * DH:(_m + 1) * DH, :] = np.eye(DH, dtype=np.float32)


def _pack_body(a_ref, b_ref, e_ref, out_ref):
    e = e_ref[0]
    a = jnp.dot(a_ref[...], e, preferred_element_type=jnp.float32).astype(jnp.bfloat16)
    nb = jnp.dot(b_ref[0:1], e, preferred_element_type=jnp.float32).astype(jnp.bfloat16)
    nxt = jnp.concatenate([a[1:], nb], axis=0)
    out_ref[...] = jnp.concatenate([a, nxt], axis=1)


def _pack(val_bf):
    return pl.pallas_call(
        _pack_body,
        grid=(B, M, TPB),
        in_specs=[
            pl.BlockSpec((T, D), lambda b, m, i: (b * TPB + i, 0)),
            pl.BlockSpec((T, D),
                         lambda b, m, i: (jnp.minimum(b * TPB + i + 1, NT - 1), 0)),
            pl.BlockSpec((1, D, DH), lambda b, m, i: (m, 0, 0)),
        ],
        out_specs=pl.BlockSpec((T, 2 * DH), lambda b, m, i: ((b * M + m) * TPB + i, 0)),
        out_shape=jax.ShapeDtypeStruct((B * M * LEN, 2 * DH), jnp.bfloat16),
    )(val_bf, val_bf, jnp.asarray(_ESEL, dtype=jnp.bfloat16))


QC = 20
NCH = RPW // QC


@functools.cache
def _sample_fn():
    mesh = plsc.VectorSubcoreMesh(core_axis_name="c", subcore_axis_name="s",
                                  num_cores=2, num_subcores=16)

    @functools.partial(
        pl.kernel,
        out_type=jax.ShapeDtypeStruct((ROWS, D), jnp.float32),
        mesh=mesh,
        scratch_types=[
            pltpu.VMEM((2, QC, 256), jnp.int32),
            pltpu.VMEM((2, QC, 512), jnp.float32),
            pltpu.VMEM((2, 256, 64), jnp.bfloat16),
            pltpu.VMEM((QC, D), jnp.float32),
            pltpu.SemaphoreType.DMA,
            pltpu.SemaphoreType.DMA,
            pltpu.SemaphoreType.DMA,
        ],
        compiler_params=pltpu.CompilerParams(use_tc_tiling_on_sc=False,
                                             needs_layout_passes=False),
    )
    def _sample(value_hbm, idx_hbm, w_hbm, out_hbm,
                idxc, wc, rowb, outc, sem_ch, sem_g0, sem_g1):
        wid = lax.axis_index("s") * 2 + lax.axis_index("c")
        base = wid * RPW

        def fire_chunk(c):
            q0 = jnp.minimum(base + c * QC, ROWS - QC)
            s = lax.rem(c, 2)
            pltpu.async_copy(idx_hbm.at[pl.ds(q0, QC)], idxc.at[s], sem_ch)
            pltpu.async_copy(w_hbm.at[pl.ds(q0, QC)], wc.at[s], sem_ch)

        def wait_chunk():
            pltpu.make_async_copy(idx_hbm.at[pl.ds(0, QC)], idxc.at[0], sem_ch).wait()
            pltpu.make_async_copy(w_hbm.at[pl.ds(0, QC)], wc.at[0], sem_ch).wait()

        def fire_row(s, r, rb, sem):
            pltpu.async_copy(value_hbm.at[idxc.at[s, r]], rowb.at[rb], sem)

        def wait_row(rb, sem):
            pltpu.make_async_copy(value_hbm.at[pl.ds(0, 256)],
                                  rowb.at[rb], sem).wait()

        def accum_row(s, r, rb):
            def mbody(m, carry):
                acc_e = jnp.zeros((16,), jnp.float32)
                acc_o = jnp.zeros((16,), jnp.float32)
                for t2 in range(2):
                    w0vec = wc[s, r, pl.ds(t2 * 128 + m * 16, 16)]
                    w1vec = wc[s, r, pl.ds(256 + t2 * 128 + m * 16, 16)]
                    for j in range(16):
                        w0 = w0vec[j]
                        w1 = w1vec[j]
                        pr = t2 * 128 + m * 16 + j
                        p0e, p0o = plsc.unpack(rowb[rb, pr, 0:32],
                                               format=plsc.PackFormat.INTERLEAVED)
                        p1e, p1o = plsc.unpack(rowb[rb, pr, 32:64],
                                               format=plsc.PackFormat.INTERLEAVED)
                        acc_e = acc_e + w0 * p0e + w1 * p1e
                        acc_o = acc_o + w0 * p0o + w1 * p1o
                outc[r, pl.ds(m * 32, 16)] = acc_e
                outc[r, pl.ds(m * 32 + 16, 16)] = acc_o
                return carry
            lax.fori_loop(0, M, mbody, 0)

        def chunk_body(c, carry):
            s = lax.rem(c, 2)
            def pair_body(p, carry2):
                fire_row(s, 2 * p + 1, 1, sem_g1)
                wait_row(0, sem_g0)
                accum_row(s, 2 * p, 0)

                @pl.when(2 * p + 2 < QC)
                def _():
                    fire_row(s, 2 * p + 2, 0, sem_g0)
                wait_row(1, sem_g1)
                accum_row(s, 2 * p + 1, 1)
                return carry2
            lax.fori_loop(0, QC // 2, pair_body, 0)
            pltpu.sync_copy(outc, out_hbm.at[pl.ds(base + c * QC, QC)])
            wait_chunk()
            fire_chunk(c + 2)
            fire_row(1 - s, 0, 0, sem_g0)
            return carry

        fire_chunk(jnp.int32(0))
        wait_chunk()
        fire_chunk(jnp.int32(1))
        fire_row(jnp.int32(0), jnp.int32(0), 0, sem_g0)
        lax.fori_loop(0, NCH, chunk_body, 0)
        wait_row(0, sem_g0)
        wait_chunk()

    return _sample


def _proj_body(o_ref, w_ref, b_ref, out_ref):
    out_ref[...] = (jnp.dot(o_ref[...], w_ref[...], preferred_element_type=jnp.float32)
                    + b_ref[...])


def _proj(o2, W_out, bout2):
    return pl.pallas_call(
        _proj_body,
        grid=(NT,),
        in_specs=[
            pl.BlockSpec((T, D), lambda i: (i, 0)),
            pl.BlockSpec((D, D), lambda i: (0, 0)),
            pl.BlockSpec((1, D), lambda i: (0, 0)),
        ],
        out_specs=pl.BlockSpec((T, D), lambda i: (i, 0)),
        out_shape=jax.ShapeDtypeStruct((ROWS, D), jnp.float32),
    )(o2, W_out, bout2)


def kernel(query, reference_points, input_flatten, input_spatial_shapes,
           input_level_start_index, W_off, b_off, W_attn, b_attn, W_val, b_val,
           W_out, b_out):
    q2 = query.reshape(ROWS, D)
    x2 = input_flatten.reshape(ROWS, D)
    rp2 = reference_points.reshape(ROWS, L * 2)
    W_offx = W_off[:, 0::2]
    W_offy = W_off[:, 1::2]
    cbx = (b_off[0::2] - 0.5).reshape(1, 128)
    cby = (b_off[1::2] - 0.5).reshape(1, 128)
    battn2 = b_attn.reshape(1, 128)
    perm = jnp.asarray(_PERM_VAL)
    W_valp = W_val[:, perm]
    bval2 = b_val[perm].reshape(1, D)

    val_bf, idx_all, w_all = _prep(
        q2, x2, rp2, W_valp, bval2, W_offx, W_offy, cbx, cby, W_attn, battn2,
        jnp.asarray(_SX), jnp.asarray(_SY), jnp.asarray(_BLK),
        jnp.asarray(_WL_I), jnp.asarray(_BASEP), jnp.asarray(_WM2), jnp.asarray(_HM1))

    tbl = _pack(val_bf)
    out_mid = _sample_fn()(tbl, idx_all, w_all)

    out = _proj(out_mid, W_out, b_out.reshape(1, D))
    return out.reshape(B, LEN, D)


# speedup vs baseline: 1.2249x; 1.2249x over previous
import functools
import numpy as np
import jax
import jax.numpy as jnp
from jax import lax
from jax.experimental import pallas as pl
from jax.experimental.pallas import tpu as pltpu
from jax.experimental.pallas import tpu_sc as plsc

B = 2
D = 256
M = 8
L = 4
P = 4
DH = 32
SHAPES = ((128, 128), (64, 64), (32, 32), (16, 16))
LEN = sum(h * w for h, w in SHAPES)
ROWS = B * LEN
T = 256
NT = ROWS // T
TPB = LEN // T
NWORK = 32
RPW = ROWS // NWORK

_lanes = np.arange(M * L * P)
_m_ln = _lanes // (L * P)
_l_ln = (_lanes % (L * P)) // P
_W_np = np.array([w for h, w in SHAPES], np.float32)
_H_np = np.array([h for h, w in SHAPES], np.float32)
_start_np = np.concatenate([[0], np.cumsum([h * w for h, w in SHAPES[:-1]])]).astype(np.int64)

_WL_I = _W_np[_l_ln].astype(np.int32).reshape(1, 128)
_BASEP = (_m_ln.astype(np.int64) * LEN + _start_np[_l_ln]).astype(np.int32).reshape(1, 128)
_WM2 = (_W_np[_l_ln] - 2.0).reshape(1, 128).astype(np.float32)
_HM1 = (_H_np[_l_ln] - 1.0).reshape(1, 128).astype(np.float32)

_SX = np.zeros((8, 128), np.float32)
_SY = np.zeros((8, 128), np.float32)
for _ln in range(128):
    _SX[2 * _l_ln[_ln] + 0, _ln] = _W_np[_l_ln[_ln]]
    _SY[2 * _l_ln[_ln] + 1, _ln] = _H_np[_l_ln[_ln]]

_BLK = (_lanes[:, None] // 16 == _lanes[None, :] // 16).astype(np.float32)


def _prep_body(q_ref, x_ref, rp_ref, wval_ref, bval_ref, woffx_ref, woffy_ref,
               cbx_ref, cby_ref, wattn_ref, battn_ref, sx_ref, sy_ref, blk_ref,
               wl_ref, basep_ref, wm2_ref, hm1_ref,
               val_out, idx_out, w_out):
    q = q_ref[...]
    v = jnp.dot(x_ref[...], wval_ref[...], preferred_element_type=jnp.float32) + bval_ref[...]
    val_out[...] = v.astype(jnp.bfloat16)

    hi = jax.lax.Precision.HIGHEST
    gx = (jnp.dot(q, woffx_ref[...], preferred_element_type=jnp.float32, precision=hi)
          + jnp.dot(rp_ref[...], sx_ref[...], preferred_element_type=jnp.float32, precision=hi)
          + cbx_ref[...])
    gy = (jnp.dot(q, woffy_ref[...], preferred_element_type=jnp.float32, precision=hi)
          + jnp.dot(rp_ref[...], sy_ref[...], preferred_element_type=jnp.float32, precision=hi)
          + cby_ref[...])

    a = jnp.dot(q, wattn_ref[...], preferred_element_type=jnp.float32, precision=hi) + battn_ref[...]
    e = jnp.exp(a - jnp.max(a, axis=1, keepdims=True))
    aw = e / jnp.dot(e, blk_ref[...], preferred_element_type=jnp.float32)

    x0 = jnp.floor(gx)
    y0 = jnp.floor(gy)
    y1 = y0 + 1.0
    fx = gx - x0
    fy = gy - y0

    wm2 = wm2_ref[...]
    hm1 = hm1_ref[...]
    x0c = jnp.clip(x0, 0.0, wm2)
    y0c = jnp.clip(y0, 0.0, hm1)
    y1c = jnp.clip(y1, 0.0, hm1)

    wxs0 = jnp.where(x0c == x0, 1.0 - fx, jnp.where(x0c == x0 + 1.0, fx, 0.0))
    wxs1 = jnp.where(x0c == x0, fx, jnp.where(x0c == x0 - 1.0, 1.0 - fx, 0.0))
    wy0 = jnp.where((y0 >= 0.0) & (y0 <= hm1), (1.0 - fy), 0.0) * aw
    wy1 = jnp.where((y1 >= 0.0) & (y1 <= hm1), fy, 0.0) * aw

    b = pl.program_id(0) // TPB
    boff = (b * (M * LEN)).astype(jnp.int32)
    kbase = basep_ref[...] + boff + x0c.astype(jnp.int32)
    wl = wl_ref[...]
    k0 = kbase + y0c.astype(jnp.int32) * wl
    k1 = kbase + y1c.astype(jnp.int32) * wl

    idx_out[...] = jnp.concatenate([k0, k1], axis=1)
    w_out[...] = jnp.concatenate([wy0 * wxs0, wy1 * wxs0, wy0 * wxs1, wy1 * wxs1], axis=1)


def _prep(q2, x2, rp2, W_val, bval2, W_offx, W_offy, cbx, cby, W_attn, battn2,
          sx, sy, blk, wl, basep, wm2, hm1):
    row_spec = lambda cols: pl.BlockSpec((T, cols), lambda i: (i, 0))
    full_spec = lambda r, c: pl.BlockSpec((r, c), lambda i: (0, 0))
    return pl.pallas_call(
        _prep_body,
        grid=(NT,),
        in_specs=[
            row_spec(D), row_spec(D), row_spec(8),
            full_spec(D, D), full_spec(1, D),
            full_spec(D, 128), full_spec(D, 128),
            full_spec(1, 128), full_spec(1, 128),
            full_spec(D, 128), full_spec(1, 128),
            full_spec(8, 128), full_spec(8, 128), full_spec(128, 128),
            full_spec(1, 128), full_spec(1, 128), full_spec(1, 128), full_spec(1, 128),
        ],
        out_specs=[row_spec(D), row_spec(256), row_spec(512)],
        out_shape=[
            jax.ShapeDtypeStruct((ROWS, D), jnp.bfloat16),
            jax.ShapeDtypeStruct((ROWS, 256), jnp.int32),
            jax.ShapeDtypeStruct((ROWS, 512), jnp.float32),
        ],
    )(q2, x2, rp2, W_val, bval2, W_offx, W_offy, cbx, cby, W_attn, battn2,
      sx, sy, blk, wl, basep, wm2, hm1)


QC = 20
NCH = RPW // QC


@functools.cache
def _sample_fn():
    mesh = plsc.VectorSubcoreMesh(core_axis_name="c", subcore_axis_name="s",
                                  num_cores=2, num_subcores=16)

    @functools.partial(
        pl.kernel,
        out_type=jax.ShapeDtypeStruct((ROWS, D), jnp.float32),
        mesh=mesh,
        scratch_types=[
            pltpu.VMEM((2, QC, 256), jnp.int32),
            pltpu.VMEM((2, QC, 512), jnp.float32),
            pltpu.VMEM((2, 256, 64), jnp.bfloat16),
            pltpu.VMEM((QC, D), jnp.float32),
            pltpu.SemaphoreType.DMA,
            pltpu.SemaphoreType.DMA,
            pltpu.SemaphoreType.DMA,
        ],
        compiler_params=pltpu.CompilerParams(use_tc_tiling_on_sc=False,
                                             needs_layout_passes=False),
    )
    def _sample(value_hbm, idx_hbm, w_hbm, out_hbm,
                idxc, wc, rowb, outc, sem_ch, sem_g0, sem_g1):
        wid = lax.axis_index("s") * 2 + lax.axis_index("c")
        base = wid * RPW

        def fire_chunk(c):
            q0 = jnp.minimum(base + c * QC, ROWS - QC)
            s = lax.rem(c, 2)
            pltpu.async_copy(idx_hbm.at[pl.ds(q0, QC)], idxc.at[s], sem_ch)
            pltpu.async_copy(w_hbm.at[pl.ds(q0, QC)], wc.at[s], sem_ch)

        def wait_chunk():
            pltpu.make_async_copy(idx_hbm.at[pl.ds(0, QC)], idxc.at[0], sem_ch).wait()
            pltpu.make_async_copy(w_hbm.at[pl.ds(0, QC)], wc.at[0], sem_ch).wait()

        def fire_row(s, r, rb, sem):
            pltpu.async_copy(value_hbm.at[idxc.at[s, r]], rowb.at[rb], sem)

        def wait_row(rb, sem):
            pltpu.make_async_copy(value_hbm.at[pl.ds(0, 256)],
                                  rowb.at[rb], sem).wait()

        def accum_row(s, r, rb):
            def mbody(m, carry):
                acc_e = jnp.zeros((16,), jnp.float32)
                acc_o = jnp.zeros((16,), jnp.float32)
                for t2 in range(2):
                    w0vec = wc[s, r, pl.ds(t2 * 128 + m * 16, 16)]
                    w1vec = wc[s, r, pl.ds(256 + t2 * 128 + m * 16, 16)]
                    for j in range(16):
                        w0 = w0vec[j]
                        w1 = w1vec[j]
                        pr = t2 * 128 + m * 16 + j
                        p0e, p0o = plsc.unpack(rowb[rb, pr, 0:32],
                                               format=plsc.PackFormat.INTERLEAVED)
                        p1e, p1o = plsc.unpack(rowb[rb, pr, 32:64],
                                               format=plsc.PackFormat.INTERLEAVED)
                        acc_e = acc_e + w0 * p0e + w1 * p1e
                        acc_o = acc_o + w0 * p0o + w1 * p1o
                outc[r, pl.ds(m * 32, 16)] = acc_e
                outc[r, pl.ds(m * 32 + 16, 16)] = acc_o
                return carry
            lax.fori_loop(0, M, mbody, 0)

        def chunk_body(c, carry):
            s = lax.rem(c, 2)
            def pair_body(p, carry2):
                fire_row(s, 2 * p + 1, 1, sem_g1)
                wait_row(0, sem_g0)
                accum_row(s, 2 * p, 0)

                @pl.when(2 * p + 2 < QC)
                def _():
                    fire_row(s, 2 * p + 2, 0, sem_g0)
                wait_row(1, sem_g1)
                accum_row(s, 2 * p + 1, 1)
                return carry2
            lax.fori_loop(0, QC // 2, pair_body, 0)
            pltpu.sync_copy(outc, out_hbm.at[pl.ds(base + c * QC, QC)])
            wait_chunk()
            fire_chunk(c + 2)
            fire_row(1 - s, 0, 0, sem_g0)
            return carry

        fire_chunk(jnp.int32(0))
        wait_chunk()
        fire_chunk(jnp.int32(1))
        fire_row(jnp.int32(0), jnp.int32(0), 0, sem_g0)
        lax.fori_loop(0, NCH, chunk_body, 0)
        wait_row(0, sem_g0)
        wait_chunk()

    return _sample


def _proj_body(o_ref, w_ref, b_ref, out_ref):
    out_ref[...] = (jnp.dot(o_ref[...], w_ref[...], preferred_element_type=jnp.float32)
                    + b_ref[...])


def _proj(o2, W_out, bout2):
    return pl.pallas_call(
        _proj_body,
        grid=(NT,),
        in_specs=[
            pl.BlockSpec((T, D), lambda i: (i, 0)),
            pl.BlockSpec((D, D), lambda i: (0, 0)),
            pl.BlockSpec((1, D), lambda i: (0, 0)),
        ],
        out_specs=pl.BlockSpec((T, D), lambda i: (i, 0)),
        out_shape=jax.ShapeDtypeStruct((ROWS, D), jnp.float32),
    )(o2, W_out, bout2)


def kernel(query, reference_points, input_flatten, input_spatial_shapes,
           input_level_start_index, W_off, b_off, W_attn, b_attn, W_val, b_val,
           W_out, b_out):
    q2 = query.reshape(ROWS, D)
    x2 = input_flatten.reshape(ROWS, D)
    rp2 = reference_points.reshape(ROWS, L * 2)
    W_offx = W_off[:, 0::2]
    W_offy = W_off[:, 1::2]
    cbx = (b_off[0::2] - 0.5).reshape(1, 128)
    cby = (b_off[1::2] - 0.5).reshape(1, 128)
    battn2 = b_attn.reshape(1, 128)
    bval2 = b_val.reshape(1, D)

    val_bf, idx_all, w_all = _prep(
        q2, x2, rp2, W_val, bval2, W_offx, W_offy, cbx, cby, W_attn, battn2,
        jnp.asarray(_SX), jnp.asarray(_SY), jnp.asarray(_BLK),
        jnp.asarray(_WL_I), jnp.asarray(_BASEP), jnp.asarray(_WM2), jnp.asarray(_HM1))

    vh = (val_bf.reshape(B, LEN, M, 2, 16).transpose(0, 2, 1, 4, 3)
          .reshape(B * M * LEN, 32))
    vh1 = jnp.concatenate([vh[1:], jnp.zeros((1, 32), jnp.bfloat16)], axis=0)
    tbl = jnp.concatenate([vh, vh1], axis=1)
    out_mid = _sample_fn()(tbl, idx_all, w_all)

    out = _proj(out_mid, W_out, b_out.reshape(1, D))
    return out.reshape(B, LEN, D)


# speedup vs baseline: 1.2382x; 1.0109x over previous
import functools
import numpy as np
import jax
import jax.numpy as jnp
from jax import lax
from jax.experimental import pallas as pl
from jax.experimental.pallas import tpu as pltpu
from jax.experimental.pallas import tpu_sc as plsc

B = 2
D = 256
M = 8
L = 4
P = 4
DH = 32
SHAPES = ((128, 128), (64, 64), (32, 32), (16, 16))
LEN = sum(h * w for h, w in SHAPES)
ROWS = B * LEN
T = 256
NT = ROWS // T
TPB = LEN // T
NWORK = 32
RPW = ROWS // NWORK

_lanes = np.arange(M * L * P)
_m_ln = _lanes // (L * P)
_l_ln = (_lanes % (L * P)) // P
_W_np = np.array([w for h, w in SHAPES], np.float32)
_H_np = np.array([h for h, w in SHAPES], np.float32)
_start_np = np.concatenate([[0], np.cumsum([h * w for h, w in SHAPES[:-1]])]).astype(np.int64)

_WL_I = _W_np[_l_ln].astype(np.int32).reshape(1, 128)
_BASEP = (_m_ln.astype(np.int64) * LEN + _start_np[_l_ln]).astype(np.int32).reshape(1, 128)
_WM2 = (_W_np[_l_ln] - 2.0).reshape(1, 128).astype(np.float32)
_HM1 = (_H_np[_l_ln] - 1.0).reshape(1, 128).astype(np.float32)

_SX = np.zeros((8, 128), np.float32)
_SY = np.zeros((8, 128), np.float32)
for _ln in range(128):
    _SX[2 * _l_ln[_ln] + 0, _ln] = _W_np[_l_ln[_ln]]
    _SY[2 * _l_ln[_ln] + 1, _ln] = _H_np[_l_ln[_ln]]

_BLK = (_lanes[:, None] // 16 == _lanes[None, :] // 16).astype(np.float32)


def _prep_body(q_ref, x_ref, rp_ref, wval_ref, bval_ref, woffx_ref, woffy_ref,
               cbx_ref, cby_ref, wattn_ref, battn_ref, sx_ref, sy_ref, blk_ref,
               wl_ref, basep_ref, wm2_ref, hm1_ref,
               val_out, idx_out, w_out):
    q = q_ref[...]
    v = jnp.dot(x_ref[...], wval_ref[...], preferred_element_type=jnp.float32) + bval_ref[...]
    val_out[...] = v.astype(jnp.bfloat16)

    hi = jax.lax.Precision.HIGHEST
    gx = (jnp.dot(q, woffx_ref[...], preferred_element_type=jnp.float32, precision=hi)
          + jnp.dot(rp_ref[...], sx_ref[...], preferred_element_type=jnp.float32, precision=hi)
          + cbx_ref[...])
    gy = (jnp.dot(q, woffy_ref[...], preferred_element_type=jnp.float32, precision=hi)
          + jnp.dot(rp_ref[...], sy_ref[...], preferred_element_type=jnp.float32, precision=hi)
          + cby_ref[...])

    a = jnp.dot(q, wattn_ref[...], preferred_element_type=jnp.float32, precision=hi) + battn_ref[...]
    e = jnp.exp(a - jnp.max(a, axis=1, keepdims=True))
    aw = e / jnp.dot(e, blk_ref[...], preferred_element_type=jnp.float32)

    x0 = jnp.floor(gx)
    y0 = jnp.floor(gy)
    y1 = y0 + 1.0
    fx = gx - x0
    fy = gy - y0

    wm2 = wm2_ref[...]
    hm1 = hm1_ref[...]
    x0c = jnp.clip(x0, 0.0, wm2)
    y0c = jnp.clip(y0, 0.0, hm1)
    y1c = jnp.clip(y1, 0.0, hm1)

    wxs0 = jnp.where(x0c == x0, 1.0 - fx, jnp.where(x0c == x0 + 1.0, fx, 0.0))
    wxs1 = jnp.where(x0c == x0, fx, jnp.where(x0c == x0 - 1.0, 1.0 - fx, 0.0))
    wy0 = jnp.where((y0 >= 0.0) & (y0 <= hm1), (1.0 - fy), 0.0) * aw
    wy1 = jnp.where((y1 >= 0.0) & (y1 <= hm1), fy, 0.0) * aw

    b = pl.program_id(0) // TPB
    boff = (b * (M * LEN)).astype(jnp.int32)
    kbase = basep_ref[...] + boff + x0c.astype(jnp.int32)
    wl = wl_ref[...]
    k0 = kbase + y0c.astype(jnp.int32) * wl
    k1 = kbase + y1c.astype(jnp.int32) * wl

    idx_out[...] = jnp.concatenate([k0, k1], axis=1)
    w_out[...] = jnp.concatenate([wy0 * wxs0, wy1 * wxs0, wy0 * wxs1, wy1 * wxs1], axis=1)


def _prep(q2, x2, rp2, W_val, bval2, W_offx, W_offy, cbx, cby, W_attn, battn2,
          sx, sy, blk, wl, basep, wm2, hm1):
    row_spec = lambda cols: pl.BlockSpec((T, cols), lambda i: (i, 0))
    full_spec = lambda r, c: pl.BlockSpec((r, c), lambda i: (0, 0))
    return pl.pallas_call(
        _prep_body,
        grid=(NT,),
        in_specs=[
            row_spec(D), row_spec(D), row_spec(8),
            full_spec(D, D), full_spec(1, D),
            full_spec(D, 128), full_spec(D, 128),
            full_spec(1, 128), full_spec(1, 128),
            full_spec(D, 128), full_spec(1, 128),
            full_spec(8, 128), full_spec(8, 128), full_spec(128, 128),
            full_spec(1, 128), full_spec(1, 128), full_spec(1, 128), full_spec(1, 128),
        ],
        out_specs=[row_spec(D), row_spec(256), row_spec(512)],
        out_shape=[
            jax.ShapeDtypeStruct((ROWS, D), jnp.bfloat16),
            jax.ShapeDtypeStruct((ROWS, 256), jnp.int32),
            jax.ShapeDtypeStruct((ROWS, 512), jnp.float32),
        ],
    )(q2, x2, rp2, W_val, bval2, W_offx, W_offy, cbx, cby, W_attn, battn2,
      sx, sy, blk, wl, basep, wm2, hm1)


QC = 40
NCH = RPW // QC


@functools.cache
def _sample_fn():
    mesh = plsc.VectorSubcoreMesh(core_axis_name="c", subcore_axis_name="s",
                                  num_cores=2, num_subcores=16)

    @functools.partial(
        pl.kernel,
        out_type=jax.ShapeDtypeStruct((ROWS, D), jnp.float32),
        mesh=mesh,
        scratch_types=[
            pltpu.VMEM((2, QC, 256), jnp.int32),
            pltpu.VMEM((2, QC, 512), jnp.float32),
            pltpu.VMEM((2, 256, 64), jnp.bfloat16),
            pltpu.VMEM((QC, D), jnp.float32),
            pltpu.SemaphoreType.DMA,
            pltpu.SemaphoreType.DMA,
            pltpu.SemaphoreType.DMA,
        ],
        compiler_params=pltpu.CompilerParams(use_tc_tiling_on_sc=False,
                                             needs_layout_passes=False),
    )
    def _sample(value_hbm, idx_hbm, w_hbm, out_hbm,
                idxc, wc, rowb, outc, sem_ch, sem_g0, sem_g1):
        wid = lax.axis_index("s") * 2 + lax.axis_index("c")
        base = wid * RPW

        def fire_chunk(c):
            q0 = jnp.minimum(base + c * QC, ROWS - QC)
            s = lax.rem(c, 2)
            pltpu.async_copy(idx_hbm.at[pl.ds(q0, QC)], idxc.at[s], sem_ch)
            pltpu.async_copy(w_hbm.at[pl.ds(q0, QC)], wc.at[s], sem_ch)

        def wait_chunk():
            pltpu.make_async_copy(idx_hbm.at[pl.ds(0, QC)], idxc.at[0], sem_ch).wait()
            pltpu.make_async_copy(w_hbm.at[pl.ds(0, QC)], wc.at[0], sem_ch).wait()

        def fire_row(s, r, rb, sem):
            pltpu.async_copy(value_hbm.at[idxc.at[s, r]], rowb.at[rb], sem)

        def wait_row(rb, sem):
            pltpu.make_async_copy(value_hbm.at[pl.ds(0, 256)],
                                  rowb.at[rb], sem).wait()

        def accum_row(s, r, rb):
            himask = jnp.full((16,), -65536, jnp.int32)

            def half(v32):
                ev = plsc.bitcast(v32 << 16, jnp.float32)
                od = plsc.bitcast(v32 & himask, jnp.float32)
                return ev, od

            def mbody(m, carry):
                acc_e = jnp.zeros((16,), jnp.float32)
                acc_o = jnp.zeros((16,), jnp.float32)
                for t2 in range(2):
                    w0vec = wc[s, r, pl.ds(t2 * 128 + m * 16, 16)]
                    w1vec = wc[s, r, pl.ds(256 + t2 * 128 + m * 16, 16)]
                    for j in range(16):
                        w0 = w0vec[j]
                        w1 = w1vec[j]
                        pr = t2 * 128 + m * 16 + j
                        p0e, p0o = half(plsc.bitcast(rowb[rb, pr, 0:32], jnp.int32))
                        p1e, p1o = half(plsc.bitcast(rowb[rb, pr, 32:64], jnp.int32))
                        acc_e = acc_e + w0 * p0e + w1 * p1e
                        acc_o = acc_o + w0 * p0o + w1 * p1o
                outc[r, pl.ds(m * 32, 16)] = acc_e
                outc[r, pl.ds(m * 32 + 16, 16)] = acc_o
                return carry
            lax.fori_loop(0, M, mbody, 0)

        def chunk_body(c, carry):
            s = lax.rem(c, 2)
            def pair_body(p, carry2):
                fire_row(s, 2 * p + 1, 1, sem_g1)
                wait_row(0, sem_g0)
                accum_row(s, 2 * p, 0)

                @pl.when(2 * p + 2 < QC)
                def _():
                    fire_row(s, 2 * p + 2, 0, sem_g0)
                wait_row(1, sem_g1)
                accum_row(s, 2 * p + 1, 1)
                return carry2
            lax.fori_loop(0, QC // 2, pair_body, 0)
            pltpu.sync_copy(outc, out_hbm.at[pl.ds(base + c * QC, QC)])
            wait_chunk()
            fire_chunk(c + 2)
            fire_row(1 - s, 0, 0, sem_g0)
            return carry

        fire_chunk(jnp.int32(0))
        wait_chunk()
        fire_chunk(jnp.int32(1))
        fire_row(jnp.int32(0), jnp.int32(0), 0, sem_g0)
        lax.fori_loop(0, NCH, chunk_body, 0)
        wait_row(0, sem_g0)
        wait_chunk()

    return _sample


def _proj_body(o_ref, w_ref, b_ref, out_ref):
    out_ref[...] = (jnp.dot(o_ref[...], w_ref[...], preferred_element_type=jnp.float32)
                    + b_ref[...])


def _proj(o2, W_out, bout2):
    return pl.pallas_call(
        _proj_body,
        grid=(NT,),
        in_specs=[
            pl.BlockSpec((T, D), lambda i: (i, 0)),
            pl.BlockSpec((D, D), lambda i: (0, 0)),
            pl.BlockSpec((1, D), lambda i: (0, 0)),
        ],
        out_specs=pl.BlockSpec((T, D), lambda i: (i, 0)),
        out_shape=jax.ShapeDtypeStruct((ROWS, D), jnp.float32),
    )(o2, W_out, bout2)


def kernel(query, reference_points, input_flatten, input_spatial_shapes,
           input_level_start_index, W_off, b_off, W_attn, b_attn, W_val, b_val,
           W_out, b_out):
    q2 = query.reshape(ROWS, D)
    x2 = input_flatten.reshape(ROWS, D)
    rp2 = reference_points.reshape(ROWS, L * 2)
    W_offx = W_off[:, 0::2]
    W_offy = W_off[:, 1::2]
    cbx = (b_off[0::2] - 0.5).reshape(1, 128)
    cby = (b_off[1::2] - 0.5).reshape(1, 128)
    battn2 = b_attn.reshape(1, 128)
    bval2 = b_val.reshape(1, D)

    val_bf, idx_all, w_all = _prep(
        q2, x2, rp2, W_val, bval2, W_offx, W_offy, cbx, cby, W_attn, battn2,
        jnp.asarray(_SX), jnp.asarray(_SY), jnp.asarray(_BLK),
        jnp.asarray(_WL_I), jnp.asarray(_BASEP), jnp.asarray(_WM2), jnp.asarray(_HM1))

    vh = (val_bf.reshape(B, LEN, M, 2, 16).transpose(0, 2, 1, 4, 3)
          .reshape(B * M * LEN, 32))
    vh1 = jnp.concatenate([vh[1:], jnp.zeros((1, 32), jnp.bfloat16)], axis=0)
    tbl = jnp.concatenate([vh, vh1], axis=1)
    out_mid = _sample_fn()(tbl, idx_all, w_all)

    out = _proj(out_mid, W_out, b_out.reshape(1, D))
    return out.reshape(B, LEN, D)
